# Initial kernel scaffold; baseline (speedup 1.0000x reference)
import jax
import jax.numpy as jnp
from jax.experimental import pallas as pl


def kernel(x, Ws):
    raise NotImplementedError("write your pallas kernel here")


# speedup vs baseline: 34.3902x; 34.3902x over previous
import jax
import jax.numpy as jnp
from jax.experimental import pallas as pl
from jax.experimental.pallas import tpu as pltpu

_SEARCH_BITS = 31


def _stage0(x_ref, w_ref, z_ref):
    z_ref[0] = jax.lax.dot_general(x_ref[0], w_ref[0], (((1,), (1,)), ((), ())),
                                   preferred_element_type=jnp.float32)


def _stage1(z_ref, sqc_ref, sqr_ref, t_ref, s_ref, scr_ref, *, k, blk):
    n = z_ref.shape[1]
    z = z_ref[0]
    sq_col = sqc_ref[0]
    sq_row = sqr_ref[0]
    acc = jnp.float32(0.0)
    for rb in range(n // blk):
        zr = z[rb * blk:(rb + 1) * blk]
        g = jax.lax.dot_general(zr, z, (((1,), (1,)), ((), ())),
                                preferred_element_type=jnp.float32)
        d2 = jnp.maximum(sq_col[rb * blk:(rb + 1) * blk] + sq_row - 2.0 * g, 0.0)
        acc = acc + jnp.sum(jnp.sqrt(d2))
        scr_ref[rb * blk:(rb + 1) * blk, :] = jax.lax.bitcast_convert_type(
            d2, jnp.int32)
    s_ref[0, 0, :] = acc * jnp.ones((128,), jnp.float32)

    kf = jnp.float32(k)

    def body(_, carry):
        lo, hi = carry
        mid = lo + ((hi - lo) >> 1)
        below = scr_ref[:] <= mid
        cnt = jnp.sum(jnp.where(below, 1.0, 0.0), axis=1, keepdims=True)
        ge = cnt >= kf
        return jnp.where(ge, lo, mid), jnp.where(ge, mid, hi)

    lo0 = jnp.full((n, 1), -1, jnp.int32)
    hi0 = jnp.full((n, 1), 0x7F800000, jnp.int32)
    _, hi = jax.lax.fori_loop(0, _SEARCH_BITS, body, (lo0, hi0))
    t_ref[0] = jax.lax.bitcast_convert_type(hi, jnp.float32)


def _stage2(z_ref, sqc_ref, sqr_ref, tc_ref, tr_ref, inv_ref, o_ref, *,
            nheads, blk):
    n = z_ref.shape[1]
    start = pl.program_id(1) * blk
    acc = jnp.zeros((blk, n), jnp.float32)
    for h in range(nheads):
        zh = z_ref[h]
        zr = z_ref[h, pl.ds(start, blk), :]
        g = jax.lax.dot_general(zr, zh, (((1,), (1,)), ((), ())),
                                preferred_element_type=jnp.float32)
        sq_r = sqc_ref[h, pl.ds(start, blk), :]
        sq_c = sqr_ref[h]
        d2 = jnp.maximum(sq_r + sq_c - 2.0 * g, 0.0)
        t_r = tc_ref[h, pl.ds(start, blk), :]
        t_c = tr_ref[h]
        thr = jnp.maximum(t_r, t_c)
        acc = acc + jnp.where(d2 <= thr,
                              jnp.exp(-d2 * inv_ref[0, h]), 0.0)
    o_ref[0] = acc * jnp.float32(1.0 / nheads)


def kernel(x, Ws):
    B, N, D = x.shape
    H, K, _ = Ws.shape
    k = max(1, int(N * 0.15))
    blk = min(256, N)

    z4 = pl.pallas_call(
        _stage0,
        grid=(B * H,),
        in_specs=[
            pl.BlockSpec((1, N, D), lambda i: (i // H, 0, 0)),
            pl.BlockSpec((1, K, D), lambda i: (i % H, 0, 0)),
        ],
        out_specs=pl.BlockSpec((1, N, K), lambda i: (i, 0, 0)),
        out_shape=jax.ShapeDtypeStruct((B * H, N, K), jnp.float32),
    )(x, Ws)

    sq = jnp.sum(z4 * z4, axis=-1)
    sq_col = sq.reshape(B * H, N, 1)
    sq_row = sq.reshape(B * H, 1, N)

    tcol, ssum = pl.pallas_call(
        lambda zr, sc, sr, tr, s, scr: _stage1(zr, sc, sr, tr, s, scr,
                                               k=k, blk=blk),
        grid=(B * H,),
        in_specs=[
            pl.BlockSpec((1, N, K), lambda i: (i, 0, 0)),
            pl.BlockSpec((1, N, 1), lambda i: (i, 0, 0)),
            pl.BlockSpec((1, 1, N), lambda i: (i, 0, 0)),
        ],
        out_specs=[
            pl.BlockSpec((1, N, 1), lambda i: (i, 0, 0)),
            pl.BlockSpec((1, 1, 128), lambda i: (i, 0, 0)),
        ],
        out_shape=[
            jax.ShapeDtypeStruct((B * H, N, 1), jnp.float32),
            jax.ShapeDtypeStruct((B * H, 1, 128), jnp.float32),
        ],
        scratch_shapes=[pltpu.VMEM((N, N), jnp.int32)],
    )(z4, sq_col, sq_row)

    sums = ssum[:, 0, 0].reshape(B, H)
    mean = jnp.sum(sums, axis=0) / jnp.float32(B * N * N)
    inv = (1.0 / (2.0 * mean * mean + 1e-8)).reshape(1, H).astype(jnp.float32)
    trow = tcol.reshape(B * H, 1, N)

    out = pl.pallas_call(
        lambda zr, sc, sr, tc, tr, iv, orf: _stage2(zr, sc, sr, tc, tr, iv,
                                                    orf, nheads=H, blk=blk),
        grid=(B, N // blk),
        in_specs=[
            pl.BlockSpec((H, N, K), lambda b, rb: (b, 0, 0)),
            pl.BlockSpec((H, N, 1), lambda b, rb: (b, 0, 0)),
            pl.BlockSpec((H, 1, N), lambda b, rb: (b, 0, 0)),
            pl.BlockSpec((H, N, 1), lambda b, rb: (b, 0, 0)),
            pl.BlockSpec((H, 1, N), lambda b, rb: (b, 0, 0)),
            pl.BlockSpec((1, H), lambda b, rb: (0, 0),
                         memory_space=pltpu.SMEM),
        ],
        out_specs=pl.BlockSpec((1, blk, N), lambda b, rb: (b, rb, 0)),
        out_shape=jax.ShapeDtypeStruct((B, N, N), jnp.float32),
    )(z4, sq_col, sq_row, tcol, trow, inv)
    return out
